# baseline (speedup 1.0000x reference)
import functools
import math

import jax
import jax.numpy as jnp
from jax import lax
from jax.experimental import pallas as pl
from jax.experimental.pallas import tpu as pltpu
from jax.experimental.pallas import tpu_sc as plsc

SCALE = math.sqrt(256.0)

_NC, _NS = 2, 16
_NW = _NC * _NS

_CHUNK = 128


def _sc_gather(table, idx2d):
    R = idx2d.shape[0]
    D = table.shape[1]
    rows_per_w = R // _NW
    N = R * _CHUNK

    mesh = plsc.VectorSubcoreMesh(core_axis_name="c", subcore_axis_name="s")

    @functools.partial(
        pl.kernel,
        out_type=jax.ShapeDtypeStruct((N, D), jnp.float32),
        mesh=mesh,
        scratch_types=[
            pltpu.VMEM((rows_per_w, _CHUNK), jnp.int32),
            pltpu.VMEM((_CHUNK, D), jnp.float32),
            pltpu.SemaphoreType.DMA,
        ],
        compiler_params=pltpu.CompilerParams(use_tc_tiling_on_sc=False),
    )
    def gather_k(table_hbm, idx_hbm, out_hbm, idx_v, rows_v, sem):
        wid = lax.axis_index("s") * _NC + lax.axis_index("c")
        row_base = wid * rows_per_w
        pltpu.sync_copy(idx_hbm.at[pl.ds(row_base, rows_per_w)], idx_v)

        def body(j, _):
            pltpu.async_copy(table_hbm.at[idx_v.at[j]], rows_v, sem).wait()
            out_base = (row_base + j) * _CHUNK
            pltpu.sync_copy(rows_v, out_hbm.at[pl.ds(out_base, _CHUNK)])
            return _

        lax.fori_loop(0, rows_per_w, body, None)

    return gather_k(table, idx2d)


def _tc_project(emb, W, b):
    N, D = emb.shape
    M = W.shape[0]
    BLK = 2048
    grid = (N // BLK,)

    def proj_k(emb_ref, w_ref, b_ref, out_ref):
        acc = lax.dot_general(
            emb_ref[...], w_ref[...],
            (((1,), (1,)), ((), ())),
            preferred_element_type=jnp.float32,
        )
        out_ref[...] = (acc + b_ref[...][None, :]) * jnp.float32(SCALE)

    return pl.pallas_call(
        proj_k,
        grid=grid,
        in_specs=[
            pl.BlockSpec((BLK, D), lambda i: (i, 0)),
            pl.BlockSpec((M, D), lambda i: (0, 0)),
            pl.BlockSpec((M,), lambda i: (0,)),
        ],
        out_specs=pl.BlockSpec((BLK, M), lambda i: (i, 0)),
        out_shape=jax.ShapeDtypeStruct((N, M), jnp.float32),
    )(emb, W, b)


def kernel(x, table, W, b):
    B, L = x.shape
    N = B * L
    idx2d = x.reshape(N // _CHUNK, _CHUNK).astype(jnp.int32)
    emb = _sc_gather(table, idx2d)
    out = _tc_project(emb, W, b)
    return out.reshape(B, L, W.shape[0])


# speedup vs baseline: 1.1949x; 1.1949x over previous
import functools
import math

import jax
import jax.numpy as jnp
from jax import lax
from jax.experimental import pallas as pl
from jax.experimental.pallas import tpu as pltpu
from jax.experimental.pallas import tpu_sc as plsc

SCALE = math.sqrt(256.0)

_NC, _NS = 2, 16
_NW = _NC * _NS

_RPD = 128
_K = 5
_NCHUNK = 4
_BLK = 2048


def _sc_gather(table, idx2d):
    R = idx2d.shape[0]
    D = table.shape[1]
    rows_per_w = R // _NW
    groups = rows_per_w // _K
    assert rows_per_w % _K == 0
    brows = _K * _RPD

    mesh = plsc.VectorSubcoreMesh(core_axis_name="c", subcore_axis_name="s")

    @functools.partial(
        pl.kernel,
        out_type=jax.ShapeDtypeStruct((R * _RPD, D), jnp.float32),
        mesh=mesh,
        scratch_types=[
            pltpu.VMEM((rows_per_w, _RPD), jnp.int32),
            pltpu.VMEM((brows, D), jnp.float32),
            pltpu.VMEM((brows, D), jnp.float32),
            pltpu.SemaphoreType.DMA,
            pltpu.SemaphoreType.DMA,
            pltpu.SemaphoreType.DMA,
            pltpu.SemaphoreType.DMA,
        ],
        compiler_params=pltpu.CompilerParams(use_tc_tiling_on_sc=False),
    )
    def gather_k(table_hbm, idx_hbm, out_hbm, idx_v, buf_a, buf_b, gsem_a,
                 gsem_b, wsem_a, wsem_b):
        wid = lax.axis_index("s") * _NC + lax.axis_index("c")
        row_base = wid * rows_per_w
        pltpu.sync_copy(idx_hbm.at[pl.ds(row_base, rows_per_w)], idx_v)

        def fire(g, buf, gsem):
            for j in range(_K):
                pltpu.async_copy(
                    table_hbm.at[idx_v.at[g * _K + j]],
                    buf.at[pl.ds(j * _RPD, _RPD)],
                    gsem,
                )

        def drain(g, buf, gsem):
            for j in range(_K):
                pltpu.make_async_copy(
                    table_hbm.at[idx_v.at[g * _K + j]],
                    buf.at[pl.ds(j * _RPD, _RPD)],
                    gsem,
                ).wait()

        def store(g, buf, wsem):
            base = (row_base + g * _K) * _RPD
            pltpu.async_copy(buf, out_hbm.at[pl.ds(base, brows)], wsem)

        def store_wait(g, buf, wsem):
            base = (row_base + g * _K) * _RPD
            pltpu.make_async_copy(
                buf, out_hbm.at[pl.ds(base, brows)], wsem).wait()

        fire(0, buf_a, gsem_a)

        def step(i, p_buf, p_wsem, p_gsem, q_buf, q_wsem, q_gsem):
            drain(i, p_buf, p_gsem)

            @pl.when(i + 1 < groups)
            def _():
                @pl.when(i >= 1)
                def _():
                    store_wait(i - 1, q_buf, q_wsem)
                fire(i + 1, q_buf, q_gsem)

            store(i, p_buf, p_wsem)

        def body(i, _):
            @pl.when(lax.rem(i, 2) == 0)
            def _():
                step(i, buf_a, wsem_a, gsem_a, buf_b, wsem_b, gsem_b)

            @pl.when(lax.rem(i, 2) == 1)
            def _():
                step(i, buf_b, wsem_b, gsem_b, buf_a, wsem_a, gsem_a)

            return _

        lax.fori_loop(0, groups, body, None)

        @pl.when(groups >= 2)
        def _():
            store_wait(groups - 2, buf_a if groups % 2 == 0 else buf_b,
                       wsem_a if groups % 2 == 0 else wsem_b)
        store_wait(groups - 1, buf_b if groups % 2 == 0 else buf_a,
                   wsem_b if groups % 2 == 0 else wsem_a)

    return gather_k(table, idx2d)


def _proj_body(emb_ref, we_ref, wo_ref, b_ref, out_ref):
    e = emb_ref[...]
    top = lax.dot_general(e, we_ref[...], (((1,), (0,)), ((), ())),
                          preferred_element_type=jnp.float32)
    bot = lax.dot_general(e, wo_ref[...], (((1,), (0,)), ((), ())),
                          preferred_element_type=jnp.float32)
    bias = b_ref[...][None, :]
    h = _BLK // 2
    out_ref[pl.ds(0, h), :] = top + bias
    out_ref[pl.ds(h, h), :] = bot + bias


def _proj_body_alias(emb_ref, we_ref, wo_ref, b_ref, _, out_ref):
    _proj_body(emb_ref, we_ref, wo_ref, b_ref, out_ref)


def _tc_project(emb2, We, Wo, bs, out_prev, chunk_idx, n_out_blocks):
    nblk = (2 * emb2.shape[0]) // _BLK
    base = chunk_idx * nblk
    body = _proj_body if out_prev is None else _proj_body_alias
    in_specs = [
        pl.BlockSpec((_BLK // 2, 128), lambda i: (i, 0)),
        pl.BlockSpec((128, 256), lambda i: (0, 0)),
        pl.BlockSpec((128, 256), lambda i: (0, 0)),
        pl.BlockSpec((256,), lambda i: (0,)),
    ]
    args = [emb2, We, Wo, bs]
    aliases = {}
    if out_prev is not None:
        in_specs.append(pl.BlockSpec(memory_space=pl.ANY))
        args.append(out_prev)
        aliases = {4: 0}
    return pl.pallas_call(
        body,
        grid=(nblk,),
        in_specs=in_specs,
        out_specs=pl.BlockSpec((_BLK, 256), lambda i, base=base: (base + i, 0)),
        out_shape=jax.ShapeDtypeStruct((n_out_blocks * _BLK, 256), jnp.float32),
        input_output_aliases=aliases,
    )(*args)


def kernel(x, table, W, b):
    B, L = x.shape
    N = B * L
    D = table.shape[1]
    M = W.shape[0]

    idx = x.reshape(N).astype(jnp.int32)
    idx_perm = (
        idx.reshape(N // _BLK, 2, _BLK // 2)
        .transpose(0, 2, 1)
        .reshape(N)
    )

    Ws = (W * jnp.float32(SCALE)).T
    z = jnp.zeros((D, M), jnp.float32)
    We = jnp.concatenate([Ws, z], axis=0)
    Wo = jnp.concatenate([z, Ws], axis=0)
    bs = b * jnp.float32(SCALE)

    n_out_blocks = N // _BLK
    chunk_tokens = N // _NCHUNK

    out = None
    for c in range(_NCHUNK):
        idx_c = lax.slice(idx_perm, (c * chunk_tokens,),
                          ((c + 1) * chunk_tokens,))
        idx2d = idx_c.reshape(chunk_tokens // _RPD, _RPD)
        emb = _sc_gather(table, idx2d)
        emb2 = emb.reshape(chunk_tokens // 2, 2 * D)
        out = _tc_project(emb2, We, Wo, bs, out, c, n_out_blocks)

    return out.reshape(B, L, M)


# speedup vs baseline: 1.3068x; 1.0937x over previous
import functools
import math

import jax
import jax.numpy as jnp
from jax import lax
from jax.experimental import pallas as pl
from jax.experimental.pallas import tpu as pltpu
from jax.experimental.pallas import tpu_sc as plsc

SCALE = math.sqrt(256.0)

_NC, _NS = 2, 16
_NW = _NC * _NS

_RPD = 128
_K = 5
_NCHUNK = 4
_BLK = 8192


def _sc_gather(table, idx2d):
    R = idx2d.shape[0]
    D = table.shape[1]
    rows_per_w = R // _NW
    groups = rows_per_w // _K
    assert rows_per_w % _K == 0
    brows = _K * _RPD

    mesh = plsc.VectorSubcoreMesh(core_axis_name="c", subcore_axis_name="s")

    @functools.partial(
        pl.kernel,
        out_type=jax.ShapeDtypeStruct((R * _RPD, D), jnp.float32),
        mesh=mesh,
        scratch_types=[
            pltpu.VMEM((rows_per_w, _RPD), jnp.int32),
            pltpu.VMEM((brows, D), jnp.float32),
            pltpu.VMEM((brows, D), jnp.float32),
            pltpu.SemaphoreType.DMA,
            pltpu.SemaphoreType.DMA,
            pltpu.SemaphoreType.DMA,
            pltpu.SemaphoreType.DMA,
        ],
        compiler_params=pltpu.CompilerParams(use_tc_tiling_on_sc=False),
    )
    def gather_k(table_hbm, idx_hbm, out_hbm, idx_v, buf_a, buf_b, gsem_a,
                 gsem_b, wsem_a, wsem_b):
        wid = lax.axis_index("s") * _NC + lax.axis_index("c")
        row_base = wid * rows_per_w
        pltpu.sync_copy(idx_hbm.at[pl.ds(row_base, rows_per_w)], idx_v)

        def fire(g, buf, gsem):
            for j in range(_K):
                pltpu.async_copy(
                    table_hbm.at[idx_v.at[g * _K + j]],
                    buf.at[pl.ds(j * _RPD, _RPD)],
                    gsem,
                )

        def drain(g, buf, gsem):
            for j in range(_K):
                pltpu.make_async_copy(
                    table_hbm.at[idx_v.at[g * _K + j]],
                    buf.at[pl.ds(j * _RPD, _RPD)],
                    gsem,
                ).wait()

        def store(g, buf, wsem):
            base = (row_base + g * _K) * _RPD
            pltpu.async_copy(buf, out_hbm.at[pl.ds(base, brows)], wsem)

        def store_wait(g, buf, wsem):
            base = (row_base + g * _K) * _RPD
            pltpu.make_async_copy(
                buf, out_hbm.at[pl.ds(base, brows)], wsem).wait()

        fire(0, buf_a, gsem_a)

        def step(i, p_buf, p_wsem, p_gsem, q_buf, q_wsem, q_gsem):
            drain(i, p_buf, p_gsem)

            @pl.when(i + 1 < groups)
            def _():
                @pl.when(i >= 1)
                def _():
                    store_wait(i - 1, q_buf, q_wsem)
                fire(i + 1, q_buf, q_gsem)

            store(i, p_buf, p_wsem)

        def body(i, _):
            @pl.when(lax.rem(i, 2) == 0)
            def _():
                step(i, buf_a, wsem_a, gsem_a, buf_b, wsem_b, gsem_b)

            @pl.when(lax.rem(i, 2) == 1)
            def _():
                step(i, buf_b, wsem_b, gsem_b, buf_a, wsem_a, gsem_a)

            return _

        lax.fori_loop(0, groups, body, None)

        @pl.when(groups >= 2)
        def _():
            store_wait(groups - 2, buf_a if groups % 2 == 0 else buf_b,
                       wsem_a if groups % 2 == 0 else wsem_b)
        store_wait(groups - 1, buf_b if groups % 2 == 0 else buf_a,
                   wsem_b if groups % 2 == 0 else wsem_a)

    return gather_k(table, idx2d)


def _proj_body(emb_ref, we_ref, wo_ref, b_ref, out_ref):
    e = emb_ref[...]
    top = lax.dot_general(e, we_ref[...], (((1,), (0,)), ((), ())),
                          preferred_element_type=jnp.float32)
    bot = lax.dot_general(e, wo_ref[...], (((1,), (0,)), ((), ())),
                          preferred_element_type=jnp.float32)
    bias = b_ref[...][None, :]
    h = _BLK // 2
    out_ref[pl.ds(0, h), :] = top + bias
    out_ref[pl.ds(h, h), :] = bot + bias


def _proj_body_alias(emb_ref, we_ref, wo_ref, b_ref, _, out_ref):
    _proj_body(emb_ref, we_ref, wo_ref, b_ref, out_ref)


def _tc_project(emb2, We, Wo, bs, out_prev, chunk_idx, n_out_blocks):
    nblk = (2 * emb2.shape[0]) // _BLK
    base = chunk_idx * nblk
    body = _proj_body if out_prev is None else _proj_body_alias
    in_specs = [
        pl.BlockSpec((_BLK // 2, 128), lambda i: (i, 0)),
        pl.BlockSpec((128, 256), lambda i: (0, 0)),
        pl.BlockSpec((128, 256), lambda i: (0, 0)),
        pl.BlockSpec((256,), lambda i: (0,)),
    ]
    args = [emb2, We, Wo, bs]
    aliases = {}
    if out_prev is not None:
        in_specs.append(pl.BlockSpec(memory_space=pl.ANY))
        args.append(out_prev)
        aliases = {4: 0}
    return pl.pallas_call(
        body,
        grid=(nblk,),
        in_specs=in_specs,
        out_specs=pl.BlockSpec((_BLK, 256), lambda i, base=base: (base + i, 0)),
        out_shape=jax.ShapeDtypeStruct((n_out_blocks * _BLK, 256), jnp.float32),
        input_output_aliases=aliases,
    )(*args)


def kernel(x, table, W, b):
    B, L = x.shape
    N = B * L
    D = table.shape[1]
    M = W.shape[0]

    idx = x.reshape(N).astype(jnp.int32)
    idx_perm = (
        idx.reshape(N // _BLK, 2, _BLK // 2)
        .transpose(0, 2, 1)
        .reshape(N)
    )

    Ws = (W * jnp.float32(SCALE)).T
    z = jnp.zeros((D, M), jnp.float32)
    We = jnp.concatenate([Ws, z], axis=0)
    Wo = jnp.concatenate([z, Ws], axis=0)
    bs = b * jnp.float32(SCALE)

    n_out_blocks = N // _BLK
    chunk_tokens = N // _NCHUNK

    out = None
    for c in range(_NCHUNK):
        idx_c = lax.slice(idx_perm, (c * chunk_tokens,),
                          ((c + 1) * chunk_tokens,))
        idx2d = idx_c.reshape(chunk_tokens // _RPD, _RPD)
        emb = _sc_gather(table, idx2d)
        emb2 = emb.reshape(chunk_tokens // 2, 2 * D)
        out = _tc_project(emb2, We, Wo, bs, out, c, n_out_blocks)

    return out.reshape(B, L, M)


# speedup vs baseline: 1.3180x; 1.0085x over previous
import functools
import math

import jax
import jax.numpy as jnp
from jax import lax
from jax.experimental import pallas as pl
from jax.experimental.pallas import tpu as pltpu
from jax.experimental.pallas import tpu_sc as plsc

SCALE = math.sqrt(256.0)

_NC, _NS = 2, 16
_NW = _NC * _NS

_RPD = 128
_K = 5
_NCHUNK = 8
_BLK = 10240


def _sc_gather(table, idx2d):
    R = idx2d.shape[0]
    D = table.shape[1]
    rows_per_w = R // _NW
    groups = rows_per_w // _K
    assert rows_per_w % _K == 0
    brows = _K * _RPD

    mesh = plsc.VectorSubcoreMesh(core_axis_name="c", subcore_axis_name="s")

    @functools.partial(
        pl.kernel,
        out_type=jax.ShapeDtypeStruct((R * _RPD, D), jnp.float32),
        mesh=mesh,
        scratch_types=[
            pltpu.VMEM((rows_per_w, _RPD), jnp.int32),
            pltpu.VMEM((brows, D), jnp.float32),
            pltpu.VMEM((brows, D), jnp.float32),
            pltpu.SemaphoreType.DMA,
            pltpu.SemaphoreType.DMA,
            pltpu.SemaphoreType.DMA,
            pltpu.SemaphoreType.DMA,
        ],
        compiler_params=pltpu.CompilerParams(use_tc_tiling_on_sc=False),
    )
    def gather_k(table_hbm, idx_hbm, out_hbm, idx_v, buf_a, buf_b, gsem_a,
                 gsem_b, wsem_a, wsem_b):
        wid = lax.axis_index("s") * _NC + lax.axis_index("c")
        row_base = wid * rows_per_w
        pltpu.sync_copy(idx_hbm.at[pl.ds(row_base, rows_per_w)], idx_v)

        def fire(g, buf, gsem):
            for j in range(_K):
                pltpu.async_copy(
                    table_hbm.at[idx_v.at[g * _K + j]],
                    buf.at[pl.ds(j * _RPD, _RPD)],
                    gsem,
                )

        def drain(g, buf, gsem):
            for j in range(_K):
                pltpu.make_async_copy(
                    table_hbm.at[idx_v.at[g * _K + j]],
                    buf.at[pl.ds(j * _RPD, _RPD)],
                    gsem,
                ).wait()

        def store(g, buf, wsem):
            base = (row_base + g * _K) * _RPD
            pltpu.async_copy(buf, out_hbm.at[pl.ds(base, brows)], wsem)

        def store_wait(g, buf, wsem):
            base = (row_base + g * _K) * _RPD
            pltpu.make_async_copy(
                buf, out_hbm.at[pl.ds(base, brows)], wsem).wait()

        fire(0, buf_a, gsem_a)

        def step(i, p_buf, p_wsem, p_gsem, q_buf, q_wsem, q_gsem):
            drain(i, p_buf, p_gsem)

            @pl.when(i + 1 < groups)
            def _():
                @pl.when(i >= 1)
                def _():
                    store_wait(i - 1, q_buf, q_wsem)
                fire(i + 1, q_buf, q_gsem)

            store(i, p_buf, p_wsem)

        def body(i, _):
            @pl.when(lax.rem(i, 2) == 0)
            def _():
                step(i, buf_a, wsem_a, gsem_a, buf_b, wsem_b, gsem_b)

            @pl.when(lax.rem(i, 2) == 1)
            def _():
                step(i, buf_b, wsem_b, gsem_b, buf_a, wsem_a, gsem_a)

            return _

        lax.fori_loop(0, groups, body, None)

        @pl.when(groups >= 2)
        def _():
            store_wait(groups - 2, buf_a if groups % 2 == 0 else buf_b,
                       wsem_a if groups % 2 == 0 else wsem_b)
        store_wait(groups - 1, buf_b if groups % 2 == 0 else buf_a,
                   wsem_b if groups % 2 == 0 else wsem_a)

    return gather_k(table, idx2d)


def _proj_body(emb_ref, we_ref, wo_ref, b_ref, out_ref):
    e = emb_ref[...]
    top = lax.dot_general(e, we_ref[...], (((1,), (0,)), ((), ())),
                          preferred_element_type=jnp.float32)
    bot = lax.dot_general(e, wo_ref[...], (((1,), (0,)), ((), ())),
                          preferred_element_type=jnp.float32)
    bias = b_ref[...][None, :]
    h = _BLK // 2
    out_ref[pl.ds(0, h), :] = top + bias
    out_ref[pl.ds(h, h), :] = bot + bias


def _proj_body_alias(emb_ref, we_ref, wo_ref, b_ref, _, out_ref):
    _proj_body(emb_ref, we_ref, wo_ref, b_ref, out_ref)


def _tc_project(emb2, We, Wo, bs, out_prev, chunk_idx, n_out_blocks):
    nblk = (2 * emb2.shape[0]) // _BLK
    base = chunk_idx * nblk
    body = _proj_body if out_prev is None else _proj_body_alias
    in_specs = [
        pl.BlockSpec((_BLK // 2, 128), lambda i: (i, 0)),
        pl.BlockSpec((128, 256), lambda i: (0, 0)),
        pl.BlockSpec((128, 256), lambda i: (0, 0)),
        pl.BlockSpec((256,), lambda i: (0,)),
    ]
    args = [emb2, We, Wo, bs]
    aliases = {}
    if out_prev is not None:
        in_specs.append(pl.BlockSpec(memory_space=pl.ANY))
        args.append(out_prev)
        aliases = {4: 0}
    return pl.pallas_call(
        body,
        grid=(nblk,),
        in_specs=in_specs,
        out_specs=pl.BlockSpec((_BLK, 256), lambda i, base=base: (base + i, 0)),
        out_shape=jax.ShapeDtypeStruct((n_out_blocks * _BLK, 256), jnp.float32),
        input_output_aliases=aliases,
    )(*args)


def kernel(x, table, W, b):
    B, L = x.shape
    N = B * L
    D = table.shape[1]
    M = W.shape[0]

    idx = x.reshape(N).astype(jnp.int32)
    idx_perm = (
        idx.reshape(N // _BLK, 2, _BLK // 2)
        .transpose(0, 2, 1)
        .reshape(N)
    )

    Ws = (W * jnp.float32(SCALE)).T
    z = jnp.zeros((D, M), jnp.float32)
    We = jnp.concatenate([Ws, z], axis=0)
    Wo = jnp.concatenate([z, Ws], axis=0)
    bs = b * jnp.float32(SCALE)

    n_out_blocks = N // _BLK
    chunk_tokens = N // _NCHUNK

    out = None
    for c in range(_NCHUNK):
        idx_c = lax.slice(idx_perm, (c * chunk_tokens,),
                          ((c + 1) * chunk_tokens,))
        idx2d = idx_c.reshape(chunk_tokens // _RPD, _RPD)
        emb = _sc_gather(table, idx2d)
        emb2 = emb.reshape(chunk_tokens // 2, 2 * D)
        out = _tc_project(emb2, We, Wo, bs, out, c, n_out_blocks)

    return out.reshape(B, L, M)


# speedup vs baseline: 1.6841x; 1.2778x over previous
import functools
import math

import jax
import jax.numpy as jnp
from jax import lax
from jax.experimental import pallas as pl
from jax.experimental.pallas import tpu as pltpu
from jax.experimental.pallas import tpu_sc as plsc

SCALE = math.sqrt(256.0)

_NC, _NS = 2, 16
_NW = _NC * _NS

_RPD = 128
_K = 5
_NCHUNK = 8
_BLK = 10240


def _sc_gather(table, idx2d):
    R = idx2d.shape[0]
    D = table.shape[1]
    rows_per_w = R // _NW
    groups = rows_per_w // _K
    assert rows_per_w % _K == 0
    brows = _K * _RPD

    mesh = plsc.VectorSubcoreMesh(core_axis_name="c", subcore_axis_name="s")

    @functools.partial(
        pl.kernel,
        out_type=jax.ShapeDtypeStruct((R * _RPD, D), jnp.float32),
        mesh=mesh,
        scratch_types=[
            pltpu.VMEM((rows_per_w, _RPD), jnp.int32),
            pltpu.VMEM((brows, D), jnp.float32),
            pltpu.VMEM((brows, D), jnp.float32),
            pltpu.SemaphoreType.DMA,
            pltpu.SemaphoreType.DMA,
            pltpu.SemaphoreType.DMA,
            pltpu.SemaphoreType.DMA,
        ],
        compiler_params=pltpu.CompilerParams(use_tc_tiling_on_sc=False),
    )
    def gather_k(table_hbm, idx_hbm, out_hbm, idx_v, buf_a, buf_b, gsem_a,
                 gsem_b, wsem_a, wsem_b):
        wid = lax.axis_index("s") * _NC + lax.axis_index("c")
        row_base = wid * rows_per_w
        pltpu.sync_copy(idx_hbm.at[pl.ds(row_base, rows_per_w)], idx_v)

        def fire(g, buf, gsem):
            for j in range(_K):
                pltpu.async_copy(
                    table_hbm.at[idx_v.at[g * _K + j]],
                    buf.at[pl.ds(j * _RPD, _RPD)],
                    gsem,
                )

        def drain(g, buf, gsem):
            for j in range(_K):
                pltpu.make_async_copy(
                    table_hbm.at[idx_v.at[g * _K + j]],
                    buf.at[pl.ds(j * _RPD, _RPD)],
                    gsem,
                ).wait()

        def store(g, buf, wsem):
            base = (row_base + g * _K) * _RPD
            pltpu.async_copy(buf, out_hbm.at[pl.ds(base, brows)], wsem)

        def store_wait(g, buf, wsem):
            base = (row_base + g * _K) * _RPD
            pltpu.make_async_copy(
                buf, out_hbm.at[pl.ds(base, brows)], wsem).wait()

        fire(0, buf_a, gsem_a)

        def step(i, p_buf, p_wsem, p_gsem, q_buf, q_wsem, q_gsem):
            drain(i, p_buf, p_gsem)

            @pl.when(i + 1 < groups)
            def _():
                @pl.when(i >= 1)
                def _():
                    store_wait(i - 1, q_buf, q_wsem)
                fire(i + 1, q_buf, q_gsem)

            store(i, p_buf, p_wsem)

        def body(i, _):
            @pl.when(lax.rem(i, 2) == 0)
            def _():
                step(i, buf_a, wsem_a, gsem_a, buf_b, wsem_b, gsem_b)

            @pl.when(lax.rem(i, 2) == 1)
            def _():
                step(i, buf_b, wsem_b, gsem_b, buf_a, wsem_a, gsem_a)

            return _

        lax.fori_loop(0, groups, body, None)

        @pl.when(groups >= 2)
        def _():
            store_wait(groups - 2, buf_a if groups % 2 == 0 else buf_b,
                       wsem_a if groups % 2 == 0 else wsem_b)
        store_wait(groups - 1, buf_b if groups % 2 == 0 else buf_a,
                   wsem_b if groups % 2 == 0 else wsem_a)

    return gather_k(table, idx2d)


def _proj_body(emb_ref, we_ref, wo_ref, b_ref, out_ref):
    e = emb_ref[...]
    top = lax.dot_general(e, we_ref[...], (((1,), (0,)), ((), ())),
                          preferred_element_type=jnp.float32)
    bot = lax.dot_general(e, wo_ref[...], (((1,), (0,)), ((), ())),
                          preferred_element_type=jnp.float32)
    bias = b_ref[...][None, :]
    h = _BLK // 2
    out_ref[pl.ds(0, h), :] = top + bias
    out_ref[pl.ds(h, h), :] = bot + bias


def _proj_body_alias(emb_ref, we_ref, wo_ref, b_ref, _, out_ref):
    _proj_body(emb_ref, we_ref, wo_ref, b_ref, out_ref)


def _tc_project(emb2, We, Wo, bs, out_prev, chunk_idx, n_out_blocks):
    nblk = (2 * emb2.shape[0]) // _BLK
    base = chunk_idx * nblk
    body = _proj_body if out_prev is None else _proj_body_alias
    in_specs = [
        pl.BlockSpec((_BLK // 2, 128), lambda i: (i, 0)),
        pl.BlockSpec((128, 256), lambda i: (0, 0)),
        pl.BlockSpec((128, 256), lambda i: (0, 0)),
        pl.BlockSpec((256,), lambda i: (0,)),
    ]
    args = [emb2, We, Wo, bs]
    aliases = {}
    if out_prev is not None:
        in_specs.append(pl.BlockSpec(memory_space=pl.ANY))
        args.append(out_prev)
        aliases = {4: 0}
    return pl.pallas_call(
        body,
        grid=(nblk,),
        in_specs=in_specs,
        out_specs=pl.BlockSpec((_BLK, 256), lambda i, base=base: (base + i, 0)),
        out_shape=jax.ShapeDtypeStruct((n_out_blocks * _BLK, 256), jnp.float32),
        input_output_aliases=aliases,
    )(*args)


_TBW = 8192


def _tc_relayout_table(tableT):
    D, V = tableT.shape
    nblk = (V + _TBW - 1) // _TBW

    def trans_k(t_ref, out_ref):
        blk = t_ref[...]
        eye = jax.lax.broadcasted_iota(jnp.int32, (D, 2 * D), 0) == \
            jax.lax.broadcasted_iota(jnp.int32, (D, 2 * D), 1)
        acc = lax.dot_general(
            blk, eye.astype(jnp.float32),
            (((0,), (0,)), ((), ())),
            preferred_element_type=jnp.float32,
        )
        out_ref[...] = acc

    return pl.pallas_call(
        trans_k,
        grid=(nblk,),
        in_specs=[pl.BlockSpec((D, _TBW), lambda i: (0, i))],
        out_specs=pl.BlockSpec((_TBW, 2 * D), lambda i: (i, 0)),
        out_shape=jax.ShapeDtypeStruct((V, 2 * D), jnp.float32),
    )(tableT)


def kernel(x, table, W, b):
    B, L = x.shape
    N = B * L
    D = table.shape[1]
    M = W.shape[0]

    V = table.shape[0]
    tableP = _tc_relayout_table(table.T)
    tableL = tableP.reshape(2 * V, D)

    idx = x.reshape(N).astype(jnp.int32)
    idx_perm = 2 * (
        idx.reshape(N // _BLK, 2, _BLK // 2)
        .transpose(0, 2, 1)
        .reshape(N)
    )

    Ws = (W * jnp.float32(SCALE)).T
    z = jnp.zeros((D, M), jnp.float32)
    We = jnp.concatenate([Ws, z], axis=0)
    Wo = jnp.concatenate([z, Ws], axis=0)
    bs = b * jnp.float32(SCALE)

    n_out_blocks = N // _BLK
    chunk_tokens = N // _NCHUNK

    out = None
    for c in range(_NCHUNK):
        idx_c = lax.slice(idx_perm, (c * chunk_tokens,),
                          ((c + 1) * chunk_tokens,))
        idx2d = idx_c.reshape(chunk_tokens // _RPD, _RPD)
        emb = _sc_gather(tableL, idx2d)
        emb2 = emb.reshape(chunk_tokens // 2, 2 * D)
        out = _tc_project(emb2, We, Wo, bs, out, c, n_out_blocks)

    return out.reshape(B, L, M)


# speedup vs baseline: 2.2361x; 1.3277x over previous
import functools
import math

import jax
import jax.numpy as jnp
from jax import lax
from jax.experimental import pallas as pl
from jax.experimental.pallas import tpu as pltpu
from jax.experimental.pallas import tpu_sc as plsc

SCALE = math.sqrt(256.0)

_NC, _NS = 2, 16
_NW = _NC * _NS

_RPD = 128
_K = 5
_NCHUNK = 8
_BLK = 10240


def _sc_gather(table, idx2d):
    R = idx2d.shape[0]
    D = table.shape[1]
    rows_per_w = R // _NW
    groups = rows_per_w // _K
    assert rows_per_w % _K == 0
    brows = _K * _RPD

    mesh = plsc.VectorSubcoreMesh(core_axis_name="c", subcore_axis_name="s")

    @functools.partial(
        pl.kernel,
        out_type=jax.ShapeDtypeStruct((R * _RPD, D), jnp.float32),
        mesh=mesh,
        scratch_types=[
            pltpu.VMEM((rows_per_w, _RPD), jnp.int32),
            pltpu.VMEM((brows, D), jnp.float32),
            pltpu.VMEM((brows, D), jnp.float32),
            pltpu.SemaphoreType.DMA,
            pltpu.SemaphoreType.DMA,
            pltpu.SemaphoreType.DMA,
            pltpu.SemaphoreType.DMA,
        ],
        compiler_params=pltpu.CompilerParams(use_tc_tiling_on_sc=False),
    )
    def gather_k(table_hbm, idx_hbm, out_hbm, idx_v, buf_a, buf_b, gsem_a,
                 gsem_b, wsem_a, wsem_b):
        wid = lax.axis_index("s") * _NC + lax.axis_index("c")
        row_base = wid * rows_per_w
        pltpu.sync_copy(idx_hbm.at[pl.ds(row_base, rows_per_w)], idx_v)

        def fire(g, buf, gsem):
            for j in range(_K):
                pltpu.async_copy(
                    table_hbm.at[idx_v.at[g * _K + j]],
                    buf.at[pl.ds(j * _RPD, _RPD)],
                    gsem,
                )

        def drain(g, buf, gsem):
            for j in range(_K):
                pltpu.make_async_copy(
                    table_hbm.at[idx_v.at[g * _K + j]],
                    buf.at[pl.ds(j * _RPD, _RPD)],
                    gsem,
                ).wait()

        def store(g, buf, wsem):
            base = (row_base + g * _K) * _RPD
            pltpu.async_copy(buf, out_hbm.at[pl.ds(base, brows)], wsem)

        def store_wait(g, buf, wsem):
            base = (row_base + g * _K) * _RPD
            pltpu.make_async_copy(
                buf, out_hbm.at[pl.ds(base, brows)], wsem).wait()

        fire(0, buf_a, gsem_a)

        def step(i, p_buf, p_wsem, p_gsem, q_buf, q_wsem, q_gsem):
            drain(i, p_buf, p_gsem)

            @pl.when(i + 1 < groups)
            def _():
                @pl.when(i >= 1)
                def _():
                    store_wait(i - 1, q_buf, q_wsem)
                fire(i + 1, q_buf, q_gsem)

            store(i, p_buf, p_wsem)

        def body(i, _):
            @pl.when(lax.rem(i, 2) == 0)
            def _():
                step(i, buf_a, wsem_a, gsem_a, buf_b, wsem_b, gsem_b)

            @pl.when(lax.rem(i, 2) == 1)
            def _():
                step(i, buf_b, wsem_b, gsem_b, buf_a, wsem_a, gsem_a)

            return _

        lax.fori_loop(0, groups, body, None)

        @pl.when(groups >= 2)
        def _():
            store_wait(groups - 2, buf_a if groups % 2 == 0 else buf_b,
                       wsem_a if groups % 2 == 0 else wsem_b)
        store_wait(groups - 1, buf_b if groups % 2 == 0 else buf_a,
                   wsem_b if groups % 2 == 0 else wsem_a)

    return gather_k(table, idx2d)


def _proj_body(emb_ref, we_ref, wo_ref, b_ref, out_ref):
    e = emb_ref[...]
    top = lax.dot_general(e, we_ref[...], (((1,), (0,)), ((), ())),
                          preferred_element_type=jnp.float32)
    bot = lax.dot_general(e, wo_ref[...], (((1,), (0,)), ((), ())),
                          preferred_element_type=jnp.float32)
    bias = b_ref[...][None, :]
    h = _BLK // 2
    out_ref[pl.ds(0, h), :] = top + bias
    out_ref[pl.ds(h, h), :] = bot + bias


def _proj_body_alias(emb_ref, we_ref, wo_ref, b_ref, _, out_ref):
    _proj_body(emb_ref, we_ref, wo_ref, b_ref, out_ref)


def _tc_project(emb2, We, Wo, bs, out_prev, chunk_idx, n_out_blocks):
    nblk = (2 * emb2.shape[0]) // _BLK
    base = chunk_idx * nblk
    body = _proj_body if out_prev is None else _proj_body_alias
    in_specs = [
        pl.BlockSpec((_BLK // 2, 128), lambda i: (i, 0)),
        pl.BlockSpec((128, 256), lambda i: (0, 0)),
        pl.BlockSpec((128, 256), lambda i: (0, 0)),
        pl.BlockSpec((256,), lambda i: (0,)),
    ]
    args = [emb2, We, Wo, bs]
    aliases = {}
    if out_prev is not None:
        in_specs.append(pl.BlockSpec(memory_space=pl.ANY))
        args.append(out_prev)
        aliases = {4: 0}
    return pl.pallas_call(
        body,
        grid=(nblk,),
        in_specs=in_specs,
        out_specs=pl.BlockSpec((_BLK, 256), lambda i, base=base: (base + i, 0)),
        out_shape=jax.ShapeDtypeStruct((n_out_blocks * _BLK, 256), jnp.float32),
        input_output_aliases=aliases,
    )(*args)


def _sc_permute_idx(idx):
    N = idx.shape[0]
    nblk = N // _BLK
    H = _BLK // 2

    mesh = plsc.VectorSubcoreMesh(core_axis_name="c", subcore_axis_name="s")
    max_blocks_per_w = (nblk + _NW - 1) // _NW

    @functools.partial(
        pl.kernel,
        out_type=jax.ShapeDtypeStruct((N,), jnp.int32),
        mesh=mesh,
        scratch_types=[
            pltpu.VMEM((H,), jnp.int32),
            pltpu.VMEM((H,), jnp.int32),
            pltpu.VMEM((_BLK,), jnp.int32),
        ],
        compiler_params=pltpu.CompilerParams(
            use_tc_tiling_on_sc=False, needs_layout_passes=False),
    )
    def perm_k(idx_hbm, out_hbm, va, vb, vo):
        wid = lax.axis_index("s") * _NC + lax.axis_index("c")
        iota = lax.iota(jnp.int32, 16)

        for t in range(max_blocks_per_w):
            b = wid + _NW * t

            @pl.when(b < nblk)
            def _():
                pltpu.sync_copy(idx_hbm.at[pl.ds(b * _BLK, H)], va)
                pltpu.sync_copy(idx_hbm.at[pl.ds(b * _BLK + H, H)], vb)

                def body(k, _):
                    base = 32 * k
                    a16 = va[pl.ds(16 * k, 16)]
                    b16 = vb[pl.ds(16 * k, 16)]
                    plsc.store_scatter(vo, [base + 2 * iota], 2 * a16)
                    plsc.store_scatter(vo, [base + 1 + 2 * iota], 2 * b16)
                    return _

                lax.fori_loop(0, H // 16, body, None)
                pltpu.sync_copy(vo, out_hbm.at[pl.ds(b * _BLK, _BLK)])

    return perm_k(idx)


_TBW = 8192


def _tc_relayout_table(tableT):
    D, V = tableT.shape
    nblk = (V + _TBW - 1) // _TBW

    def trans_k(t_ref, out_ref):
        blk = t_ref[...]
        eye = jax.lax.broadcasted_iota(jnp.int32, (D, 2 * D), 0) == \
            jax.lax.broadcasted_iota(jnp.int32, (D, 2 * D), 1)
        acc = lax.dot_general(
            blk, eye.astype(jnp.float32),
            (((0,), (0,)), ((), ())),
            preferred_element_type=jnp.float32,
        )
        out_ref[...] = acc

    return pl.pallas_call(
        trans_k,
        grid=(nblk,),
        in_specs=[pl.BlockSpec((D, _TBW), lambda i: (0, i))],
        out_specs=pl.BlockSpec((_TBW, 2 * D), lambda i: (i, 0)),
        out_shape=jax.ShapeDtypeStruct((V, 2 * D), jnp.float32),
    )(tableT)


def kernel(x, table, W, b):
    B, L = x.shape
    N = B * L
    D = table.shape[1]
    M = W.shape[0]

    V = table.shape[0]
    tableP = _tc_relayout_table(table.T)
    tableL = tableP.reshape(2 * V, D)

    idx = x.reshape(N).astype(jnp.int32)
    idx_perm = _sc_permute_idx(idx)

    Ws = (W * jnp.float32(SCALE)).T
    z = jnp.zeros((D, M), jnp.float32)
    We = jnp.concatenate([Ws, z], axis=0)
    Wo = jnp.concatenate([z, Ws], axis=0)
    bs = b * jnp.float32(SCALE)

    n_out_blocks = N // _BLK
    chunk_tokens = N // _NCHUNK

    out = None
    for c in range(_NCHUNK):
        idx_c = lax.slice(idx_perm, (c * chunk_tokens,),
                          ((c + 1) * chunk_tokens,))
        idx2d = idx_c.reshape(chunk_tokens // _RPD, _RPD)
        emb = _sc_gather(tableL, idx2d)
        emb2 = emb.reshape(chunk_tokens // 2, 2 * D)
        out = _tc_project(emb2, We, Wo, bs, out, c, n_out_blocks)

    return out.reshape(B, L, M)
